# baseline (speedup 1.0000x reference)
import functools

import jax
import jax.numpy as jnp
from jax import lax
from jax.experimental import pallas as pl
from jax.experimental.pallas import tpu as pltpu
from jax.experimental.pallas import tpu_sc as plsc

B, S, D = 4, 4096, 1024
BS = 2048

_NC, _NS = 2, 16
_NW = _NC * _NS
_SEQ_W = S // _NW
_R = 32
_CHUNKS = _SEQ_W // _R
_CH = _R * D
_NVEC = _CH // 16


def _tc_body(x_ref, pe_ref, o_ref):
    o_ref[0] = x_ref[0] + pe_ref[...]


def _tc_kernel(x, pos_emb):
    grid = (S // BS, B)
    return pl.pallas_call(
        _tc_body,
        grid=grid,
        in_specs=[
            pl.BlockSpec((1, BS, D), lambda s, b: (b, s, 0)),
            pl.BlockSpec((BS, D), lambda s, b: (s, 0)),
        ],
        out_specs=pl.BlockSpec((1, BS, D), lambda s, b: (b, s, 0)),
        out_shape=jax.ShapeDtypeStruct((B, S, D), x.dtype),
    )(x, pos_emb)


def _sc_body(x_hbm, pe_hbm, out_hbm, xbuf, pebuf):
    wid = lax.axis_index("s") * _NC + lax.axis_index("c")
    seq0 = wid * _SEQ_W
    for c in range(_CHUNKS):
        row0 = seq0 + c * _R
        pltpu.sync_copy(pe_hbm.at[pl.ds(row0 * D, _CH)], pebuf)
        for b in range(B):
            off = b * S * D
            pltpu.sync_copy(x_hbm.at[pl.ds(off + row0 * D, _CH)], xbuf)

            @pl.loop(0, _NVEC, unroll=8)
            def _add(i):
                sl = pl.ds(i * 16, 16)
                plsc.addupdate(xbuf.at[sl], pebuf[sl])

            pltpu.sync_copy(xbuf, out_hbm.at[pl.ds(off + row0 * D, _CH)])


def _sc_kernel(x, pos_emb):
    mesh = plsc.VectorSubcoreMesh(core_axis_name="c", subcore_axis_name="s")
    out = pl.kernel(
        _sc_body,
        out_type=jax.ShapeDtypeStruct((B * S * D,), jnp.float32),
        mesh=mesh,
        scratch_types=[
            pltpu.VMEM((_CH,), jnp.float32),
            pltpu.VMEM((_CH,), jnp.float32),
        ],
    )(x.reshape(-1), pos_emb.reshape(-1))
    return out.reshape(B, S, D)


def kernel(x, pos_emb):
    return _sc_kernel(x, pos_emb)


# speedup vs baseline: 1.1495x; 1.1495x over previous
import functools

import jax
import jax.numpy as jnp
from jax import lax
from jax.experimental import pallas as pl
from jax.experimental.pallas import tpu as pltpu
from jax.experimental.pallas import tpu_sc as plsc

B, S, D = 4, 4096, 1024
BS = 2048

_NC, _NS = 2, 16
_NW = _NC * _NS
_SEQ_W = S // _NW
_R = 32
_CHUNKS = _SEQ_W // _R
_CH = _R * D
_NVEC = _CH // 16


def _tc_body(x_ref, pe_ref, o_ref):
    o_ref[0] = x_ref[0] + pe_ref[...]


def _tc_kernel(x, pos_emb):
    grid = (S // BS, B)
    return pl.pallas_call(
        _tc_body,
        grid=grid,
        in_specs=[
            pl.BlockSpec((1, BS, D), lambda s, b: (b, s, 0)),
            pl.BlockSpec((BS, D), lambda s, b: (s, 0)),
        ],
        out_specs=pl.BlockSpec((1, BS, D), lambda s, b: (b, s, 0)),
        out_shape=jax.ShapeDtypeStruct((B, S, D), x.dtype),
    )(x, pos_emb)


def _sc_body(x_hbm, pe_hbm, out_hbm, xb0, xb1, pebuf, si0, si1, so0, so1):
    xb = (xb0, xb1)
    si = (si0, si1)
    so = (so0, so1)
    wid = lax.axis_index("s") * _NC + lax.axis_index("c")
    seq0 = wid * _SEQ_W
    pieces = [(c, b) for c in range(_CHUNKS) for b in range(B)]

    def x_in(c, b):
        return x_hbm.at[pl.ds(b * S * D + (seq0 + c * _R) * D, _CH)]

    def x_out(c, b):
        return out_hbm.at[pl.ds(b * S * D + (seq0 + c * _R) * D, _CH)]

    h_in = [None, None]
    h_out = [None, None]
    h_in[0] = pltpu.async_copy(x_in(*pieces[0]), xb[0], si[0])
    for idx, (c, b) in enumerate(pieces):
        p = idx % 2
        if b == 0:
            pltpu.sync_copy(pe_hbm.at[pl.ds((seq0 + c * _R) * D, _CH)], pebuf)
        h_in[p].wait()
        if idx + 1 < len(pieces):
            q = (idx + 1) % 2
            if h_out[q] is not None:
                h_out[q].wait()
            h_in[q] = pltpu.async_copy(x_in(*pieces[idx + 1]), xb[q], si[q])

        @pl.loop(0, _NVEC, unroll=8)
        def _add(i):
            sl = pl.ds(i * 16, 16)
            plsc.addupdate(xb[p].at[sl], pebuf[sl])

        h_out[p] = pltpu.async_copy(xb[p], x_out(c, b), so[p])
    h_out[0].wait()
    h_out[1].wait()


def _sc_kernel(x, pos_emb):
    mesh = plsc.VectorSubcoreMesh(core_axis_name="c", subcore_axis_name="s")
    out = pl.kernel(
        _sc_body,
        out_type=jax.ShapeDtypeStruct((B * S * D,), jnp.float32),
        mesh=mesh,
        scratch_types=[
            pltpu.VMEM((_CH,), jnp.float32),
            pltpu.VMEM((_CH,), jnp.float32),
            pltpu.VMEM((_CH,), jnp.float32),
            pltpu.SemaphoreType.DMA,
            pltpu.SemaphoreType.DMA,
            pltpu.SemaphoreType.DMA,
            pltpu.SemaphoreType.DMA,
        ],
    )(x.reshape(-1), pos_emb.reshape(-1))
    return out.reshape(B, S, D)


def kernel(x, pos_emb):
    return _sc_kernel(x, pos_emb)


# speedup vs baseline: 1.4451x; 1.2571x over previous
import functools

import jax
import jax.numpy as jnp
from jax import lax
from jax.experimental import pallas as pl
from jax.experimental.pallas import tpu as pltpu
from jax.experimental.pallas import tpu_sc as plsc

B, S, D = 4, 4096, 1024
BS = 2048

_NC, _NS = 2, 16
_NW = _NC * _NS
_SEQ_W = S // _NW
_R = 32
_CHUNKS = _SEQ_W // _R
_CH = _R * D
_NVEC = _CH // 16


def _tc_body(x_ref, pe_ref, o_ref):
    o_ref[0] = x_ref[0] + pe_ref[...]


def _tc_kernel(x, pos_emb):
    grid = (S // BS, B)
    return pl.pallas_call(
        _tc_body,
        grid=grid,
        in_specs=[
            pl.BlockSpec((1, BS, D), lambda s, b: (b, s, 0)),
            pl.BlockSpec((BS, D), lambda s, b: (s, 0)),
        ],
        out_specs=pl.BlockSpec((1, BS, D), lambda s, b: (b, s, 0)),
        out_shape=jax.ShapeDtypeStruct((B, S, D), x.dtype),
    )(x, pos_emb)


def _sc_body(x_hbm, pe_hbm, out_hbm, xb0, xb1, pebuf, si0, si1, so0, so1):
    xb = (xb0, xb1)
    si = (si0, si1)
    so = (so0, so1)
    wid = lax.axis_index("s") * _NC + lax.axis_index("c")
    seq0 = wid * _SEQ_W
    pieces = [(c, b) for c in range(_CHUNKS) for b in range(B)]

    h_in = [None, None]
    h_out = [None, None]

    def start_in(idx, q):
        c, b = pieces[idx]
        return pltpu.async_copy(
            x_hbm.at[b, pl.ds(seq0 + c * _R, _R)], xb[q], si[q]
        )

    h_in[0] = start_in(0, 0)
    for idx, (c, b) in enumerate(pieces):
        p = idx % 2
        if b == 0:
            pltpu.sync_copy(pe_hbm.at[pl.ds(seq0 + c * _R, _R)], pebuf)
        h_in[p].wait()
        if idx + 1 < len(pieces):
            q = (idx + 1) % 2
            if h_out[q] is not None:
                h_out[q].wait()
            h_in[q] = start_in(idx + 1, q)

        @pl.loop(0, _R)
        def _row(r):
            @pl.loop(0, D // 16, unroll=8)
            def _add(i):
                sl = pl.ds(i * 16, 16)
                plsc.addupdate(xb[p].at[r, sl], pebuf[r, sl])

        h_out[p] = pltpu.async_copy(
            xb[p], out_hbm.at[b, pl.ds(seq0 + c * _R, _R)], so[p]
        )
    h_out[0].wait()
    h_out[1].wait()


def _sc_kernel(x, pos_emb):
    mesh = plsc.VectorSubcoreMesh(core_axis_name="c", subcore_axis_name="s")
    out = pl.kernel(
        _sc_body,
        out_type=jax.ShapeDtypeStruct((B, S, D), jnp.float32),
        mesh=mesh,
        scratch_types=[
            pltpu.VMEM((_R, D), jnp.float32),
            pltpu.VMEM((_R, D), jnp.float32),
            pltpu.VMEM((_R, D), jnp.float32),
            pltpu.SemaphoreType.DMA,
            pltpu.SemaphoreType.DMA,
            pltpu.SemaphoreType.DMA,
            pltpu.SemaphoreType.DMA,
        ],
        compiler_params=pltpu.CompilerParams(use_tc_tiling_on_sc=True),
    )(x, pos_emb)
    return out


def kernel(x, pos_emb):
    return _sc_kernel(x, pos_emb)
